# Initial kernel scaffold; baseline (speedup 1.0000x reference)
import jax
import jax.numpy as jnp
from jax.experimental import pallas as pl


def kernel(x, table):
    raise NotImplementedError("write your pallas kernel here")


# speedup vs baseline: 3.0954x; 3.0954x over previous
import jax
import jax.numpy as jnp
from jax.experimental import pallas as pl
from jax.experimental.pallas import tpu as pltpu
from jax.experimental.pallas import tpu_sc as plsc

EMBED = 128
WINDOW = 128


def kernel(x, table):
    batch, hist = x.shape
    num_idx = batch * hist
    idx = x.reshape(1, num_idx).astype(jnp.int32)

    mesh = plsc.VectorSubcoreMesh(
        core_axis_name="core", subcore_axis_name="subcore"
    )

    @pl.kernel(
        out_type=jax.ShapeDtypeStruct((num_idx, EMBED), table.dtype),
        mesh=mesh,
    )
    def gather_kernel(tab_hbm, i_hbm, o_hbm):
        def body(i_vmem, o_vmem):
            pltpu.sync_copy(tab_hbm.at[i_vmem.at[0]], o_vmem)

        pltpu.emit_pipeline(
            body,
            grid=(num_idx // WINDOW,),
            in_specs=[
                pl.BlockSpec((1, WINDOW), index_map=lambda i: (0, i))
            ],
            out_specs=[
                pl.BlockSpec((WINDOW, EMBED), index_map=lambda i: (i, 0))
            ],
            core_axis_name=("core", "subcore"),
            dimension_semantics=(pltpu.PARALLEL,),
        )(i_hbm, o_hbm)

    out = gather_kernel(table, idx)
    return out.reshape(batch, hist, EMBED)


# speedup vs baseline: 3.2855x; 1.0614x over previous
import jax
import jax.numpy as jnp
from jax.experimental import pallas as pl
from jax.experimental.pallas import tpu as pltpu
from jax.experimental.pallas import tpu_sc as plsc

EMBED = 128
WINDOW = 256


def kernel(x, table):
    batch, hist = x.shape
    num_idx = batch * hist
    idx = x.reshape(1, num_idx).astype(jnp.int32)

    mesh = plsc.VectorSubcoreMesh(
        core_axis_name="core", subcore_axis_name="subcore"
    )

    @pl.kernel(
        out_type=jax.ShapeDtypeStruct((num_idx, EMBED), table.dtype),
        mesh=mesh,
    )
    def gather_kernel(tab_hbm, i_hbm, o_hbm):
        def body(i_vmem, o_vmem):
            pltpu.sync_copy(tab_hbm.at[i_vmem.at[0]], o_vmem)

        pltpu.emit_pipeline(
            body,
            grid=(num_idx // WINDOW,),
            in_specs=[
                pl.BlockSpec((1, WINDOW), index_map=lambda i: (0, i))
            ],
            out_specs=[
                pl.BlockSpec((WINDOW, EMBED), index_map=lambda i: (i, 0))
            ],
            core_axis_name=("core", "subcore"),
            dimension_semantics=(pltpu.PARALLEL,),
        )(i_hbm, o_hbm)

    out = gather_kernel(table, idx)
    return out.reshape(batch, hist, EMBED)


# speedup vs baseline: 5.8967x; 1.7948x over previous
import functools

import jax
import jax.numpy as jnp
from jax import lax
from jax.experimental import pallas as pl
from jax.experimental.pallas import tpu as pltpu
from jax.experimental.pallas import tpu_sc as plsc

NC = 2
NS = 16
NW = NC * NS
EMBED = 128
CB = 8


def kernel(x, table):
    batch, hist = x.shape
    idx = x.reshape(batch * hist).astype(jnp.int32)
    rows_per_worker = batch // NW
    n_chunks = rows_per_worker // CB
    chunk_idx = CB * hist

    mesh = plsc.VectorSubcoreMesh(core_axis_name="c", subcore_axis_name="s")

    @functools.partial(
        pl.kernel,
        mesh=mesh,
        out_type=jax.ShapeDtypeStruct((batch, hist, EMBED), table.dtype),
        scratch_types=[
            pltpu.VMEM((chunk_idx,), jnp.int32),
            pltpu.VMEM((chunk_idx,), jnp.int32),
            pltpu.VMEM((chunk_idx, EMBED), table.dtype),
            pltpu.VMEM((chunk_idx, EMBED), table.dtype),
            pltpu.SemaphoreType.DMA,
            pltpu.SemaphoreType.DMA,
            pltpu.SemaphoreType.DMA,
            pltpu.SemaphoreType.DMA,
        ],
    )
    def embed_kernel(
        tab_hbm, idx_hbm, out_hbm, i0, i1, r0, r1, g0, g1, o0, o1
    ):
        idx_v = (i0, i1)
        rows_v = (r0, r1)
        gsem = (g0, g1)
        osem = (o0, o1)
        wid = lax.axis_index("c") * NS + lax.axis_index("s")
        base_row = wid * rows_per_worker

        def issue(c, b):
            off = (base_row + c * CB) * hist
            pltpu.sync_copy(idx_hbm.at[pl.ds(off, chunk_idx)], idx_v[b])
            pltpu.async_copy(tab_hbm.at[idx_v[b]], rows_v[b], gsem[b])

        def wait_gather(b):
            pltpu.make_async_copy(
                tab_hbm.at[idx_v[b]], rows_v[b], gsem[b]
            ).wait()

        def fire_out(c, b):
            for j in range(CB):
                row = base_row + c * CB + j
                pltpu.async_copy(
                    rows_v[b].at[pl.ds(j * hist, hist)],
                    out_hbm.at[row],
                    osem[b],
                )

        def drain_out(b):
            for j in range(CB):
                pltpu.make_async_copy(
                    rows_v[b].at[pl.ds(j * hist, hist)],
                    out_hbm.at[base_row],
                    osem[b],
                ).wait()

        issue(0, 0)
        issue(1, 1)

        @pl.loop(0, n_chunks, step=2)
        def _(c0):
            for b in range(2):
                c = c0 + b
                wait_gather(b)
                fire_out(c, b)

                @pl.when(c + 2 < n_chunks)
                def _():
                    drain_out(b)
                    issue(c + 2, b)

        drain_out(0)
        drain_out(1)

    return embed_kernel(table, idx)


# speedup vs baseline: 5.9078x; 1.0019x over previous
import functools

import jax
import jax.numpy as jnp
from jax import lax
from jax.experimental import pallas as pl
from jax.experimental.pallas import tpu as pltpu
from jax.experimental.pallas import tpu_sc as plsc

NC = 2
NS = 16
NW = NC * NS
EMBED = 128
CB = 4
NBUF = 4


def kernel(x, table):
    batch, hist = x.shape
    idx = x.reshape(batch * hist).astype(jnp.int32)
    rows_per_worker = batch // NW
    n_chunks = rows_per_worker // CB
    chunk_idx = CB * hist
    worker_idx = rows_per_worker * hist

    mesh = plsc.VectorSubcoreMesh(core_axis_name="c", subcore_axis_name="s")

    @functools.partial(
        pl.kernel,
        mesh=mesh,
        out_type=jax.ShapeDtypeStruct((batch, hist, EMBED), table.dtype),
        scratch_types=[
            pltpu.VMEM((worker_idx,), jnp.int32),
            pltpu.VMEM((chunk_idx, EMBED), table.dtype),
            pltpu.VMEM((chunk_idx, EMBED), table.dtype),
            pltpu.VMEM((chunk_idx, EMBED), table.dtype),
            pltpu.VMEM((chunk_idx, EMBED), table.dtype),
            pltpu.SemaphoreType.DMA,
            pltpu.SemaphoreType.DMA,
            pltpu.SemaphoreType.DMA,
            pltpu.SemaphoreType.DMA,
            pltpu.SemaphoreType.DMA,
            pltpu.SemaphoreType.DMA,
            pltpu.SemaphoreType.DMA,
            pltpu.SemaphoreType.DMA,
        ],
    )
    def embed_kernel(
        tab_hbm, idx_hbm, out_hbm, idx_v,
        r0, r1, r2, r3, g0, g1, g2, g3, o0, o1, o2, o3,
    ):
        rows_v = (r0, r1, r2, r3)
        gsem = (g0, g1, g2, g3)
        osem = (o0, o1, o2, o3)
        wid = lax.axis_index("c") * NS + lax.axis_index("s")
        base_row = wid * rows_per_worker

        pltpu.sync_copy(
            idx_hbm.at[pl.ds(base_row * hist, worker_idx)], idx_v
        )

        def idx_slice(c):
            return idx_v.at[pl.ds(c * chunk_idx, chunk_idx)]

        def issue(c, b):
            pltpu.async_copy(tab_hbm.at[idx_slice(c)], rows_v[b], gsem[b])

        def wait_gather(c, b):
            pltpu.make_async_copy(
                tab_hbm.at[idx_slice(c)], rows_v[b], gsem[b]
            ).wait()

        def fire_out(c, b):
            for j in range(CB):
                row = base_row + c * CB + j
                pltpu.async_copy(
                    rows_v[b].at[pl.ds(j * hist, hist)],
                    out_hbm.at[row],
                    osem[b],
                )

        def drain_out(b):
            for j in range(CB):
                pltpu.make_async_copy(
                    rows_v[b].at[pl.ds(j * hist, hist)],
                    out_hbm.at[base_row],
                    osem[b],
                ).wait()

        for b in range(NBUF):
            issue(b, b)

        @pl.loop(0, n_chunks, step=NBUF)
        def _(c0):
            for b in range(NBUF):
                c = c0 + b
                wait_gather(c, b)
                fire_out(c, b)

                @pl.when(c + NBUF < n_chunks)
                def _():
                    drain_out(b)
                    issue(c + NBUF, b)

        for b in range(NBUF):
            drain_out(b)

    return embed_kernel(table, idx)
